# Initial kernel scaffold; baseline (speedup 1.0000x reference)
import functools

import jax
import jax.numpy as jnp
from jax import lax
from jax.experimental import pallas as pl
from jax.experimental.pallas import tpu as pltpu
from jax.experimental.pallas import tpu_sc as plsc

N_NODES = 10000
N_EDGES = 320000
D_FEAT = 128
D_EDGE = 16
MSG_DIM = 128

NC = 2
NS = 16
NW = NC * NS
LANES = 16

EDGES_PER_W = N_EDGES // NW
CHUNK = 80
N_CHUNKS = EDGES_PER_W // CHUNK
ROWS_PER_TILE = N_NODES // NS
VEC_PER_ROW = D_FEAT // LANES


def _pq_body(nf_ref, wa_ref, wb_ref, b1_ref, p_ref, q_ref):
    nf = nf_ref[...]
    p_ref[...] = jnp.dot(nf, wa_ref[...], preferred_element_type=jnp.float32)
    q_ref[...] = (
        jnp.dot(nf, wb_ref[...], preferred_element_type=jnp.float32)
        + b1_ref[...][None, :]
    )


def _pq(node_features, w1a, w1b, b1):
    blk = 2000
    grid = (N_NODES // blk,)
    return pl.pallas_call(
        _pq_body,
        grid=grid,
        in_specs=[
            pl.BlockSpec((blk, D_FEAT), lambda i: (i, 0)),
            pl.BlockSpec((D_FEAT, MSG_DIM), lambda i: (0, 0)),
            pl.BlockSpec((D_FEAT, MSG_DIM), lambda i: (0, 0)),
            pl.BlockSpec((MSG_DIM,), lambda i: (0,)),
        ],
        out_specs=[
            pl.BlockSpec((blk, MSG_DIM), lambda i: (i, 0)),
            pl.BlockSpec((blk, MSG_DIM), lambda i: (i, 0)),
        ],
        out_shape=[
            jax.ShapeDtypeStruct((N_NODES, MSG_DIM), jnp.float32),
            jax.ShapeDtypeStruct((N_NODES, MSG_DIM), jnp.float32),
        ],
    )(node_features, w1a, w1b, b1)


def _e_body(ef_ref, wc_ref, e_ref):
    e_ref[...] = jnp.dot(
        ef_ref[...], wc_ref[...], preferred_element_type=jnp.float32
    )


def _e_proj(edge_features, w1c):
    blk = 8000
    grid = (N_EDGES // blk,)
    return pl.pallas_call(
        _e_body,
        grid=grid,
        in_specs=[
            pl.BlockSpec((blk, D_EDGE), lambda i: (i, 0)),
            pl.BlockSpec((D_EDGE, MSG_DIM), lambda i: (0, 0)),
        ],
        out_specs=pl.BlockSpec((blk, MSG_DIM), lambda i: (i, 0)),
        out_shape=jax.ShapeDtypeStruct((N_EDGES, MSG_DIM), jnp.float32),
    )(edge_features, w1c)


def _final_body(s_ref, c_ref, w2_ref, b2_ref, out_ref):
    s = s_ref[0] + s_ref[1]
    c = c_ref[0, :, 0] + c_ref[1, :, 0]
    mean = s / jnp.maximum(c, 1.0)[:, None]
    out = jnp.dot(mean, w2_ref[...], preferred_element_type=jnp.float32)
    out_ref[...] = out + jnp.where(c > 0.0, 1.0, 0.0)[:, None] * b2_ref[...][None, :]


def _final(s_partial, c_partial, W2, b2):
    blk = 2000
    grid = (N_NODES // blk,)
    return pl.pallas_call(
        _final_body,
        grid=grid,
        in_specs=[
            pl.BlockSpec((NC, blk, MSG_DIM), lambda i: (0, i, 0)),
            pl.BlockSpec((NC, blk, LANES), lambda i: (0, i, 0)),
            pl.BlockSpec((MSG_DIM, MSG_DIM), lambda i: (0, 0)),
            pl.BlockSpec((MSG_DIM,), lambda i: (0,)),
        ],
        out_specs=pl.BlockSpec((blk, MSG_DIM), lambda i: (i, 0)),
        out_shape=jax.ShapeDtypeStruct((N_NODES, MSG_DIM), jnp.float32),
    )(s_partial, c_partial, W2, b2)


def _sc_body(src_hbm, tgt_hbm, p_hbm, q_hbm, e_hbm,
             s_out, c_out,
             idx_s, idx_t, p_v, q_v, e_v, ones_v, zer_v,
             s_sh, c_sh, sem):
    cid = lax.axis_index("c")
    sid = lax.axis_index("s")
    wid = sid * NC + cid

    def fill_pq_zero(r, _):
        for k in range(VEC_PER_ROW):
            e_v[r, pl.ds(k * LANES, LANES)] = jnp.zeros((LANES,), jnp.float32)
        return 0
    lax.fori_loop(0, CHUNK, fill_pq_zero, 0)

    def fill_small(r, _):
        ones_v[r, :] = jnp.ones((LANES,), jnp.float32)
        return 0
    lax.fori_loop(0, CHUNK, fill_small, 0)

    def fill_zer(r, _):
        zer_v[r, :] = jnp.zeros((LANES,), jnp.float32)
        return 0
    lax.fori_loop(0, ROWS_PER_TILE, fill_zer, 0)

    row0 = sid * ROWS_PER_TILE
    for j in range(7):
        pltpu.sync_copy(e_v, s_sh.at[pl.ds(row0 + j * CHUNK, CHUNK)])
    pltpu.sync_copy(e_v.at[pl.ds(0, 65)], s_sh.at[pl.ds(row0 + 560, 65)])
    pltpu.sync_copy(zer_v, c_sh.at[pl.ds(row0, ROWS_PER_TILE)])
    plsc.subcore_barrier()

    base0 = wid * EDGES_PER_W

    def step(g, _):
        base = base0 + g * CHUNK
        pltpu.sync_copy(src_hbm.at[pl.ds(base, CHUNK)], idx_s)
        pltpu.sync_copy(tgt_hbm.at[pl.ds(base, CHUNK)], idx_t)
        pltpu.async_copy(p_hbm.at[idx_s], p_v, sem).wait()
        pltpu.async_copy(q_hbm.at[idx_t], q_v, sem).wait()
        pltpu.sync_copy(e_hbm.at[pl.ds(base, CHUNK)], e_v)

        def row(r, _):
            for k in range(VEC_PER_ROW):
                sl = pl.ds(k * LANES, LANES)
                p_v[r, sl] = jnp.maximum(p_v[r, sl] + q_v[r, sl] + e_v[r, sl], 0.0)
            return 0
        lax.fori_loop(0, CHUNK, row, 0)

        pltpu.sync_copy(p_v, s_sh.at[idx_t], add=True)
        pltpu.sync_copy(ones_v, c_sh.at[idx_t], add=True)
        return 0

    lax.fori_loop(0, N_CHUNKS, step, 0)
    plsc.subcore_barrier()

    pltpu.sync_copy(s_sh.at[pl.ds(row0, ROWS_PER_TILE)],
                    s_out.at[cid].at[pl.ds(row0, ROWS_PER_TILE)])
    pltpu.sync_copy(c_sh.at[pl.ds(row0, ROWS_PER_TILE)],
                    c_out.at[cid].at[pl.ds(row0, ROWS_PER_TILE)])


@functools.partial(
    pl.kernel,
    out_type=[
        jax.ShapeDtypeStruct((NC, N_NODES, MSG_DIM), jnp.float32),
        jax.ShapeDtypeStruct((NC, N_NODES, LANES), jnp.float32),
    ],
    mesh=plsc.VectorSubcoreMesh(core_axis_name="c", subcore_axis_name="s"),
    scratch_types=[
        pltpu.VMEM((CHUNK,), jnp.int32),
        pltpu.VMEM((CHUNK,), jnp.int32),
        pltpu.VMEM((CHUNK, MSG_DIM), jnp.float32),
        pltpu.VMEM((CHUNK, MSG_DIM), jnp.float32),
        pltpu.VMEM((CHUNK, MSG_DIM), jnp.float32),
        pltpu.VMEM((CHUNK, LANES), jnp.float32),
        pltpu.VMEM((ROWS_PER_TILE, LANES), jnp.float32),
        pltpu.VMEM_SHARED((N_NODES, MSG_DIM), jnp.float32),
        pltpu.VMEM_SHARED((N_NODES, LANES), jnp.float32),
        pltpu.SemaphoreType.DMA,
    ],
)
def _sc_aggregate(src_hbm, tgt_hbm, p_hbm, q_hbm, e_hbm, s_out, c_out,
                  idx_s, idx_t, p_v, q_v, e_v, ones_v, zer_v,
                  s_sh, c_sh, sem):
    _sc_body(src_hbm, tgt_hbm, p_hbm, q_hbm, e_hbm, s_out, c_out,
             idx_s, idx_t, p_v, q_v, e_v, ones_v, zer_v, s_sh, c_sh, sem)


def kernel(source_nodes, target_nodes, edge_features, node_features,
           timestamps, W1, b1, W2, b2):
    del timestamps
    w1a = W1[:D_FEAT]
    w1b = W1[D_FEAT:2 * D_FEAT]
    w1c = W1[2 * D_FEAT:]
    P, Q = _pq(node_features, w1a, w1b, b1)
    E = _e_proj(edge_features, w1c)
    s_partial, c_partial = _sc_aggregate(
        source_nodes.astype(jnp.int32), target_nodes.astype(jnp.int32),
        P, Q, E)
    return _final(s_partial, c_partial, W2, b2)


# speedup vs baseline: 3.0967x; 3.0967x over previous
import functools

import jax
import jax.numpy as jnp
from jax import lax
from jax.experimental import pallas as pl
from jax.experimental.pallas import tpu as pltpu
from jax.experimental.pallas import tpu_sc as plsc

N_NODES = 10000
N_EDGES = 320000
D_FEAT = 128
D_EDGE = 16
MSG_DIM = 128

NC = 2
NS = 16
LANES = 16

EDGES_PER_TILE = N_EDGES // (NC * NS)
CHUNK = 80
N_CHUNKS = EDGES_PER_TILE // CHUNK
ROWS_PER_TILE = 624
REM_ROWS = N_NODES - NS * ROWS_PER_TILE


def _pq_body(nf_ref, wa_ref, wb_ref, b1_ref, p_ref, q_ref):
    nf = nf_ref[...]
    p_ref[...] = jnp.dot(nf, wa_ref[...], preferred_element_type=jnp.float32)
    q_ref[...] = (
        jnp.dot(nf, wb_ref[...], preferred_element_type=jnp.float32)
        + b1_ref[...][None, :]
    )


def _pq(node_features, w1a, w1b, b1):
    blk = 2000
    grid = (N_NODES // blk,)
    return pl.pallas_call(
        _pq_body,
        grid=grid,
        in_specs=[
            pl.BlockSpec((blk, D_FEAT), lambda i: (i, 0)),
            pl.BlockSpec((D_FEAT, MSG_DIM), lambda i: (0, 0)),
            pl.BlockSpec((D_FEAT, MSG_DIM), lambda i: (0, 0)),
            pl.BlockSpec((MSG_DIM,), lambda i: (0,)),
        ],
        out_specs=[
            pl.BlockSpec((blk, MSG_DIM), lambda i: (i, 0)),
            pl.BlockSpec((blk, MSG_DIM), lambda i: (i, 0)),
        ],
        out_shape=[
            jax.ShapeDtypeStruct((N_NODES, MSG_DIM), jnp.float32),
            jax.ShapeDtypeStruct((N_NODES, MSG_DIM), jnp.float32),
        ],
    )(node_features, w1a, w1b, b1)


def _e_body(ef_ref, wc_ref, e_ref):
    e_ref[...] = jnp.dot(
        ef_ref[...], wc_ref[...], preferred_element_type=jnp.float32
    )


def _e_proj(edge_features, w1c):
    blk = 8000
    grid = (N_EDGES // blk,)
    return pl.pallas_call(
        _e_body,
        grid=grid,
        in_specs=[
            pl.BlockSpec((blk, D_EDGE), lambda i: (i, 0)),
            pl.BlockSpec((D_EDGE, MSG_DIM), lambda i: (0, 0)),
        ],
        out_specs=pl.BlockSpec((blk, MSG_DIM), lambda i: (i, 0)),
        out_shape=jax.ShapeDtypeStruct((N_EDGES, MSG_DIM), jnp.float32),
    )(edge_features, w1c)


def _final_body(s0_ref, s1_ref, c0_ref, c1_ref, w2_ref, b2_ref, out_ref):
    s = s0_ref[...] + s1_ref[...]
    c = c0_ref[:, 0] + c1_ref[:, 0]
    mean = s / jnp.maximum(c, 1.0)[:, None]
    out = jnp.dot(mean, w2_ref[...], preferred_element_type=jnp.float32)
    out_ref[...] = out + jnp.where(c > 0.0, 1.0, 0.0)[:, None] * b2_ref[...][None, :]


def _final(s_partial, c_partial, W2, b2):
    blk = 2000
    grid = (N_NODES // blk,)
    return pl.pallas_call(
        _final_body,
        grid=grid,
        in_specs=[
            pl.BlockSpec((blk, MSG_DIM), lambda i: (i, 0)),
            pl.BlockSpec((blk, MSG_DIM), lambda i: (i, 0)),
            pl.BlockSpec((blk, MSG_DIM), lambda i: (i, 0)),
            pl.BlockSpec((blk, MSG_DIM), lambda i: (i, 0)),
            pl.BlockSpec((MSG_DIM, MSG_DIM), lambda i: (0, 0)),
            pl.BlockSpec((MSG_DIM,), lambda i: (0,)),
        ],
        out_specs=pl.BlockSpec((blk, MSG_DIM), lambda i: (i, 0)),
        out_shape=jax.ShapeDtypeStruct((N_NODES, MSG_DIM), jnp.float32),
    )(
        s_partial[:N_NODES], s_partial[N_NODES:],
        c_partial[:N_NODES], c_partial[N_NODES:],
        W2, b2,
    )


@functools.partial(
    pl.kernel,
    out_type=[
        jax.ShapeDtypeStruct((NC * N_NODES, MSG_DIM), jnp.float32),
        jax.ShapeDtypeStruct((NC * N_NODES, MSG_DIM), jnp.float32),
    ],
    mesh=plsc.VectorSubcoreMesh(core_axis_name="c", subcore_axis_name="s"),
    scratch_types=[
        pltpu.VMEM((CHUNK,), jnp.int32),
        pltpu.VMEM((CHUNK,), jnp.int32),
        pltpu.VMEM((CHUNK, MSG_DIM), jnp.float32),
        pltpu.VMEM((CHUNK, MSG_DIM), jnp.float32),
        pltpu.VMEM((CHUNK, MSG_DIM), jnp.float32),
        pltpu.VMEM_SHARED((N_NODES, MSG_DIM), jnp.float32),
        pltpu.SemaphoreType.DMA,
    ],
)
def _sc_aggregate(src_hbm, tgt_hbm, p_hbm, q_hbm, e_hbm, s_out, c_out,
                  idx_s, idx_t, p_v, q_v, e_v, s_sh, sem):
    cid = lax.axis_index("c")
    sid = lax.axis_index("s")

    def fill(ref, val):
        def body(r, _):
            for k in range(MSG_DIM // LANES):
                ref[r, pl.ds(k * LANES, LANES)] = jnp.full((LANES,), val,
                                                           jnp.float32)
            return 0
        lax.fori_loop(0, CHUNK, body, 0)

    row0 = sid * ROWS_PER_TILE
    rem0 = NS * ROWS_PER_TILE

    def zero_share(zbuf):
        for j in range(7):
            pltpu.sync_copy(zbuf, s_sh.at[pl.ds(row0 + j * CHUNK, CHUNK)])
        pltpu.sync_copy(zbuf.at[pl.ds(0, 64)], s_sh.at[pl.ds(row0 + 560, 64)])

        @pl.when(sid == NS - 1)
        def _zero_rem():
            pltpu.sync_copy(zbuf.at[pl.ds(0, REM_ROWS)],
                            s_sh.at[pl.ds(rem0, REM_ROWS)])

    def copy_out(dst):
        out0 = cid * N_NODES + row0
        pltpu.sync_copy(s_sh.at[pl.ds(row0, ROWS_PER_TILE)],
                        dst.at[pl.ds(out0, ROWS_PER_TILE)])

        @pl.when(sid == NS - 1)
        def _copy_rem():
            orem = cid * N_NODES + rem0
            pltpu.sync_copy(s_sh.at[pl.ds(rem0, REM_ROWS)],
                            dst.at[pl.ds(orem, REM_ROWS)])

    fill(e_v, 0.0)
    zero_share(e_v)
    plsc.subcore_barrier()

    base0 = (cid * NS + sid) * EDGES_PER_TILE

    def step(g, _):
        base = base0 + g * CHUNK
        pltpu.sync_copy(src_hbm.at[pl.ds(base, CHUNK)], idx_s)
        pltpu.sync_copy(tgt_hbm.at[pl.ds(base, CHUNK)], idx_t)
        pltpu.async_copy(p_hbm.at[idx_s], p_v, sem).wait()
        pltpu.async_copy(q_hbm.at[idx_t], q_v, sem).wait()
        pltpu.sync_copy(e_hbm.at[pl.ds(base, CHUNK)], e_v)

        def row(r, _):
            for k in range(MSG_DIM // LANES):
                sl = pl.ds(k * LANES, LANES)
                p_v[r, sl] = jnp.maximum(p_v[r, sl] + q_v[r, sl] + e_v[r, sl],
                                         0.0)
            return 0
        lax.fori_loop(0, CHUNK, row, 0)

        pltpu.sync_copy(p_v, s_sh.at[idx_t], add=True)
        return 0

    lax.fori_loop(0, N_CHUNKS, step, 0)
    plsc.subcore_barrier()
    copy_out(s_out)

    fill(e_v, 0.0)
    zero_share(e_v)
    fill(p_v, 1.0)
    plsc.subcore_barrier()

    def step_cnt(g, _):
        base = base0 + g * CHUNK
        pltpu.sync_copy(tgt_hbm.at[pl.ds(base, CHUNK)], idx_t)
        pltpu.sync_copy(p_v, s_sh.at[idx_t], add=True)
        return 0

    lax.fori_loop(0, N_CHUNKS, step_cnt, 0)
    plsc.subcore_barrier()
    copy_out(c_out)


def kernel(source_nodes, target_nodes, edge_features, node_features,
           timestamps, W1, b1, W2, b2):
    del timestamps
    w1a = W1[:D_FEAT]
    w1b = W1[D_FEAT:2 * D_FEAT]
    w1c = W1[2 * D_FEAT:]
    P, Q = _pq(node_features, w1a, w1b, b1)
    E = _e_proj(edge_features, w1c)
    s_partial, c_partial = _sc_aggregate(
        source_nodes.astype(jnp.int32), target_nodes.astype(jnp.int32),
        P, Q, E)
    return _final(s_partial, c_partial, W2, b2)


# speedup vs baseline: 3.1007x; 1.0013x over previous
import functools

import jax
import jax.numpy as jnp
from jax import lax
from jax.experimental import pallas as pl
from jax.experimental.pallas import tpu as pltpu
from jax.experimental.pallas import tpu_sc as plsc

N_NODES = 10000
N_EDGES = 320000
D_FEAT = 128
D_EDGE = 16
MSG_DIM = 128

NC = 2
NS = 16
LANES = 16

EDGES_PER_TILE = N_EDGES // (NC * NS)
CHUNK = 80
N_CHUNKS = EDGES_PER_TILE // CHUNK
ROWS_PER_TILE = 624
REM_ROWS = N_NODES - NS * ROWS_PER_TILE


def _pq_body(nf_ref, wa_ref, wb_ref, b1_ref, p_ref, q_ref):
    nf = nf_ref[...]
    p_ref[...] = jnp.dot(nf, wa_ref[...], preferred_element_type=jnp.float32)
    q_ref[...] = (
        jnp.dot(nf, wb_ref[...], preferred_element_type=jnp.float32)
        + b1_ref[...][None, :]
    )


def _pq(node_features, w1a, w1b, b1):
    blk = 2000
    grid = (N_NODES // blk,)
    return pl.pallas_call(
        _pq_body,
        grid=grid,
        in_specs=[
            pl.BlockSpec((blk, D_FEAT), lambda i: (i, 0)),
            pl.BlockSpec((D_FEAT, MSG_DIM), lambda i: (0, 0)),
            pl.BlockSpec((D_FEAT, MSG_DIM), lambda i: (0, 0)),
            pl.BlockSpec((MSG_DIM,), lambda i: (0,)),
        ],
        out_specs=[
            pl.BlockSpec((blk, MSG_DIM), lambda i: (i, 0)),
            pl.BlockSpec((blk, MSG_DIM), lambda i: (i, 0)),
        ],
        out_shape=[
            jax.ShapeDtypeStruct((N_NODES, MSG_DIM), jnp.float32),
            jax.ShapeDtypeStruct((N_NODES, MSG_DIM), jnp.float32),
        ],
    )(node_features, w1a, w1b, b1)


def _e_body(ef_ref, wc_ref, e_ref):
    e_ref[...] = jnp.dot(
        ef_ref[...], wc_ref[...], preferred_element_type=jnp.float32
    )


def _e_proj(edge_features, w1c):
    blk = 8000
    grid = (N_EDGES // blk,)
    return pl.pallas_call(
        _e_body,
        grid=grid,
        in_specs=[
            pl.BlockSpec((blk, D_EDGE), lambda i: (i, 0)),
            pl.BlockSpec((D_EDGE, MSG_DIM), lambda i: (0, 0)),
        ],
        out_specs=pl.BlockSpec((blk, MSG_DIM), lambda i: (i, 0)),
        out_shape=jax.ShapeDtypeStruct((N_EDGES, MSG_DIM), jnp.float32),
    )(edge_features, w1c)


def _final_body(s0_ref, s1_ref, c0_ref, c1_ref, w2_ref, b2_ref, out_ref):
    s = s0_ref[...] + s1_ref[...]
    c = c0_ref[:, 0] + c1_ref[:, 0]
    mean = s / jnp.maximum(c, 1.0)[:, None]
    out = jnp.dot(mean, w2_ref[...], preferred_element_type=jnp.float32)
    out_ref[...] = out + jnp.where(c > 0.0, 1.0, 0.0)[:, None] * b2_ref[...][None, :]


def _final(s_partial, c_partial, W2, b2):
    blk = 2000
    grid = (N_NODES // blk,)
    return pl.pallas_call(
        _final_body,
        grid=grid,
        in_specs=[
            pl.BlockSpec((blk, MSG_DIM), lambda i: (i, 0)),
            pl.BlockSpec((blk, MSG_DIM), lambda i: (i, 0)),
            pl.BlockSpec((blk, MSG_DIM), lambda i: (i, 0)),
            pl.BlockSpec((blk, MSG_DIM), lambda i: (i, 0)),
            pl.BlockSpec((MSG_DIM, MSG_DIM), lambda i: (0, 0)),
            pl.BlockSpec((MSG_DIM,), lambda i: (0,)),
        ],
        out_specs=pl.BlockSpec((blk, MSG_DIM), lambda i: (i, 0)),
        out_shape=jax.ShapeDtypeStruct((N_NODES, MSG_DIM), jnp.float32),
    )(
        s_partial[:N_NODES], s_partial[N_NODES:],
        c_partial[:N_NODES], c_partial[N_NODES:],
        W2, b2,
    )


@functools.partial(
    pl.kernel,
    out_type=[
        jax.ShapeDtypeStruct((NC * N_NODES, MSG_DIM), jnp.float32),
        jax.ShapeDtypeStruct((NC * N_NODES, MSG_DIM), jnp.float32),
    ],
    mesh=plsc.VectorSubcoreMesh(core_axis_name="c", subcore_axis_name="s"),
    scratch_types=[
        pltpu.VMEM((CHUNK,), jnp.int32),
        pltpu.VMEM((CHUNK,), jnp.int32),
        pltpu.VMEM((CHUNK, MSG_DIM), jnp.float32),
        pltpu.VMEM((CHUNK, MSG_DIM), jnp.float32),
        pltpu.VMEM((CHUNK, MSG_DIM), jnp.float32),
        pltpu.VMEM_SHARED((N_NODES, MSG_DIM), jnp.float32),
        pltpu.SemaphoreType.DMA,
    ],
)
def _sc_aggregate(src_hbm, tgt_hbm, p_hbm, q_hbm, e_hbm, s_out, c_out,
                  idx_s, idx_t, p_v, q_v, e_v, s_sh, sem):
    cid = lax.axis_index("c")
    sid = lax.axis_index("s")

    def fill(ref, val):
        @plsc.parallel_loop(0, CHUNK, unroll=4)
        def body(r):
            for k in range(MSG_DIM // LANES):
                ref[r, pl.ds(k * LANES, LANES)] = jnp.full((LANES,), val,
                                                           jnp.float32)

    row0 = sid * ROWS_PER_TILE
    rem0 = NS * ROWS_PER_TILE

    def zero_share(zbuf):
        for j in range(7):
            pltpu.sync_copy(zbuf, s_sh.at[pl.ds(row0 + j * CHUNK, CHUNK)])
        pltpu.sync_copy(zbuf.at[pl.ds(0, 64)], s_sh.at[pl.ds(row0 + 560, 64)])

        @pl.when(sid == NS - 1)
        def _zero_rem():
            pltpu.sync_copy(zbuf.at[pl.ds(0, REM_ROWS)],
                            s_sh.at[pl.ds(rem0, REM_ROWS)])

    def copy_out(dst):
        out0 = cid * N_NODES + row0
        pltpu.sync_copy(s_sh.at[pl.ds(row0, ROWS_PER_TILE)],
                        dst.at[pl.ds(out0, ROWS_PER_TILE)])

        @pl.when(sid == NS - 1)
        def _copy_rem():
            orem = cid * N_NODES + rem0
            pltpu.sync_copy(s_sh.at[pl.ds(rem0, REM_ROWS)],
                            dst.at[pl.ds(orem, REM_ROWS)])

    fill(e_v, 0.0)
    zero_share(e_v)
    plsc.subcore_barrier()

    base0 = (cid * NS + sid) * EDGES_PER_TILE

    def step(g, _):
        base = base0 + g * CHUNK
        pltpu.sync_copy(src_hbm.at[pl.ds(base, CHUNK)], idx_s)
        pltpu.sync_copy(tgt_hbm.at[pl.ds(base, CHUNK)], idx_t)
        pltpu.async_copy(p_hbm.at[idx_s], p_v, sem).wait()
        pltpu.async_copy(q_hbm.at[idx_t], q_v, sem).wait()
        pltpu.sync_copy(e_hbm.at[pl.ds(base, CHUNK)], e_v)

        @plsc.parallel_loop(0, CHUNK, unroll=4)
        def row(r):
            for k in range(MSG_DIM // LANES):
                sl = pl.ds(k * LANES, LANES)
                p_v[r, sl] = jnp.maximum(p_v[r, sl] + q_v[r, sl] + e_v[r, sl],
                                         0.0)

        pltpu.sync_copy(p_v, s_sh.at[idx_t], add=True)
        return 0

    lax.fori_loop(0, N_CHUNKS, step, 0)
    plsc.subcore_barrier()
    copy_out(s_out)

    fill(e_v, 0.0)
    zero_share(e_v)
    fill(p_v, 1.0)
    plsc.subcore_barrier()

    def step_cnt(g, _):
        base = base0 + g * CHUNK
        pltpu.sync_copy(tgt_hbm.at[pl.ds(base, CHUNK)], idx_t)
        pltpu.sync_copy(p_v, s_sh.at[idx_t], add=True)
        return 0

    lax.fori_loop(0, N_CHUNKS, step_cnt, 0)
    plsc.subcore_barrier()
    copy_out(c_out)


def kernel(source_nodes, target_nodes, edge_features, node_features,
           timestamps, W1, b1, W2, b2):
    del timestamps
    w1a = W1[:D_FEAT]
    w1b = W1[D_FEAT:2 * D_FEAT]
    w1c = W1[2 * D_FEAT:]
    P, Q = _pq(node_features, w1a, w1b, b1)
    E = _e_proj(edge_features, w1c)
    s_partial, c_partial = _sc_aggregate(
        source_nodes.astype(jnp.int32), target_nodes.astype(jnp.int32),
        P, Q, E)
    return _final(s_partial, c_partial, W2, b2)


# speedup vs baseline: 4.1501x; 1.3384x over previous
import functools

import jax
import jax.numpy as jnp
from jax import lax
from jax.experimental import pallas as pl
from jax.experimental.pallas import tpu as pltpu
from jax.experimental.pallas import tpu_sc as plsc

N_NODES = 10000
N_EDGES = 320000
D_FEAT = 128
D_EDGE = 16
MSG_DIM = 128

NC = 2
NS = 16
LANES = 16

EDGES_PER_TILE = N_EDGES // (NC * NS)
CHUNK = 40
N_CHUNKS = EDGES_PER_TILE // CHUNK
N_PAIRS = N_CHUNKS // 2
ROWS_PER_TILE = 624
REM_ROWS = N_NODES - NS * ROWS_PER_TILE


def _pq_body(nf_ref, wa_ref, wb_ref, b1_ref, p_ref, q_ref):
    nf = nf_ref[...]
    p_ref[...] = jnp.dot(nf, wa_ref[...], preferred_element_type=jnp.float32)
    q_ref[...] = (
        jnp.dot(nf, wb_ref[...], preferred_element_type=jnp.float32)
        + b1_ref[...][None, :]
    )


def _pq(node_features, w1a, w1b, b1):
    blk = 2000
    grid = (N_NODES // blk,)
    return pl.pallas_call(
        _pq_body,
        grid=grid,
        in_specs=[
            pl.BlockSpec((blk, D_FEAT), lambda i: (i, 0)),
            pl.BlockSpec((D_FEAT, MSG_DIM), lambda i: (0, 0)),
            pl.BlockSpec((D_FEAT, MSG_DIM), lambda i: (0, 0)),
            pl.BlockSpec((MSG_DIM,), lambda i: (0,)),
        ],
        out_specs=[
            pl.BlockSpec((blk, MSG_DIM), lambda i: (i, 0)),
            pl.BlockSpec((blk, MSG_DIM), lambda i: (i, 0)),
        ],
        out_shape=[
            jax.ShapeDtypeStruct((N_NODES, MSG_DIM), jnp.float32),
            jax.ShapeDtypeStruct((N_NODES, MSG_DIM), jnp.float32),
        ],
    )(node_features, w1a, w1b, b1)


def _e_body(ef_ref, wc_ref, e_ref):
    e_ref[...] = jnp.dot(
        ef_ref[...], wc_ref[...], preferred_element_type=jnp.float32
    )


def _e_proj(edge_features, w1c):
    blk = 8000
    grid = (N_EDGES // blk,)
    return pl.pallas_call(
        _e_body,
        grid=grid,
        in_specs=[
            pl.BlockSpec((blk, D_EDGE), lambda i: (i, 0)),
            pl.BlockSpec((D_EDGE, MSG_DIM), lambda i: (0, 0)),
        ],
        out_specs=pl.BlockSpec((blk, MSG_DIM), lambda i: (i, 0)),
        out_shape=jax.ShapeDtypeStruct((N_EDGES, MSG_DIM), jnp.float32),
    )(edge_features, w1c)


def _final_body(s0_ref, s1_ref, c0_ref, c1_ref, w2_ref, b2_ref, out_ref):
    s = s0_ref[...] + s1_ref[...]
    c = c0_ref[:, 0] + c1_ref[:, 0]
    mean = s / jnp.maximum(c, 1.0)[:, None]
    out = jnp.dot(mean, w2_ref[...], preferred_element_type=jnp.float32)
    out_ref[...] = out + jnp.where(c > 0.0, 1.0, 0.0)[:, None] * b2_ref[...][None, :]


def _final(s_partial, c_partial, W2, b2):
    blk = 2000
    grid = (N_NODES // blk,)
    return pl.pallas_call(
        _final_body,
        grid=grid,
        in_specs=[
            pl.BlockSpec((blk, MSG_DIM), lambda i: (i, 0)),
            pl.BlockSpec((blk, MSG_DIM), lambda i: (i, 0)),
            pl.BlockSpec((blk, MSG_DIM), lambda i: (i, 0)),
            pl.BlockSpec((blk, MSG_DIM), lambda i: (i, 0)),
            pl.BlockSpec((MSG_DIM, MSG_DIM), lambda i: (0, 0)),
            pl.BlockSpec((MSG_DIM,), lambda i: (0,)),
        ],
        out_specs=pl.BlockSpec((blk, MSG_DIM), lambda i: (i, 0)),
        out_shape=jax.ShapeDtypeStruct((N_NODES, MSG_DIM), jnp.float32),
    )(
        s_partial[:N_NODES], s_partial[N_NODES:],
        c_partial[:N_NODES], c_partial[N_NODES:],
        W2, b2,
    )


@functools.partial(
    pl.kernel,
    out_type=[
        jax.ShapeDtypeStruct((NC * N_NODES, MSG_DIM), jnp.float32),
        jax.ShapeDtypeStruct((NC * N_NODES, MSG_DIM), jnp.float32),
    ],
    mesh=plsc.VectorSubcoreMesh(core_axis_name="c", subcore_axis_name="s"),
    scratch_types=[
        pltpu.VMEM((2, CHUNK), jnp.int32),
        pltpu.VMEM((2, CHUNK), jnp.int32),
        pltpu.VMEM((2, CHUNK, MSG_DIM), jnp.float32),
        pltpu.VMEM((2, CHUNK, MSG_DIM), jnp.float32),
        pltpu.VMEM((2, CHUNK, MSG_DIM), jnp.float32),
        pltpu.VMEM_SHARED((N_NODES, MSG_DIM), jnp.float32),
        pltpu.SemaphoreType.DMA,
    ],
)
def _sc_aggregate(src_hbm, tgt_hbm, p_hbm, q_hbm, e_hbm, s_out, c_out,
                  idx_s2, idx_t2, p_v2, q_v2, e_v2, s_sh, sem):
    cid = lax.axis_index("c")
    sid = lax.axis_index("s")

    def fill(ref3, b, val):
        @plsc.parallel_loop(0, CHUNK, unroll=4)
        def body(r):
            for k in range(MSG_DIM // LANES):
                ref3[b, r, pl.ds(k * LANES, LANES)] = jnp.full(
                    (LANES,), val, jnp.float32)

    row0 = sid * ROWS_PER_TILE
    rem0 = NS * ROWS_PER_TILE

    def zero_share(zbuf):
        for j in range(15):
            pltpu.sync_copy(zbuf, s_sh.at[pl.ds(row0 + j * CHUNK, CHUNK)])
        pltpu.sync_copy(zbuf.at[pl.ds(0, 24)], s_sh.at[pl.ds(row0 + 600, 24)])

        @pl.when(sid == NS - 1)
        def _zero_rem():
            pltpu.sync_copy(zbuf.at[pl.ds(0, REM_ROWS)],
                            s_sh.at[pl.ds(rem0, REM_ROWS)])

    def copy_out(dst):
        out0 = cid * N_NODES + row0
        pltpu.sync_copy(s_sh.at[pl.ds(row0, ROWS_PER_TILE)],
                        dst.at[pl.ds(out0, ROWS_PER_TILE)])

        @pl.when(sid == NS - 1)
        def _copy_rem():
            orem = cid * N_NODES + rem0
            pltpu.sync_copy(s_sh.at[pl.ds(rem0, REM_ROWS)],
                            dst.at[pl.ds(orem, REM_ROWS)])

    base0 = (cid * NS + sid) * EDGES_PER_TILE

    fill(e_v2, 0, 0.0)
    zero_share(e_v2.at[0])
    plsc.subcore_barrier()

    def fire(g, b):
        base = base0 + g * CHUNK
        pltpu.sync_copy(src_hbm.at[pl.ds(base, CHUNK)], idx_s2.at[b])
        pltpu.sync_copy(tgt_hbm.at[pl.ds(base, CHUNK)], idx_t2.at[b])
        pltpu.async_copy(p_hbm.at[idx_s2.at[b]], p_v2.at[b], sem)
        pltpu.async_copy(q_hbm.at[idx_t2.at[b]], q_v2.at[b], sem)
        pltpu.async_copy(e_hbm.at[pl.ds(base, CHUNK)], e_v2.at[b], sem)

    def drain(b):
        pltpu.make_async_copy(p_hbm.at[pl.ds(0, CHUNK)], p_v2.at[b], sem).wait()
        pltpu.make_async_copy(q_hbm.at[pl.ds(0, CHUNK)], q_v2.at[b], sem).wait()
        pltpu.make_async_copy(e_hbm.at[pl.ds(0, CHUNK)], e_v2.at[b], sem).wait()

    def consume(g, b):
        drain(b)

        @plsc.parallel_loop(0, CHUNK, unroll=4)
        def row(r):
            for k in range(MSG_DIM // LANES):
                sl = pl.ds(k * LANES, LANES)
                p_v2[b, r, sl] = jnp.maximum(
                    p_v2[b, r, sl] + q_v2[b, r, sl] + e_v2[b, r, sl], 0.0)

        pltpu.sync_copy(p_v2.at[b], s_sh.at[idx_t2.at[b]], add=True)

    fire(0, 0)

    def pair(h, _):
        g0 = 2 * h
        fire(g0 + 1, 1)
        consume(g0, 0)

        @pl.when(g0 + 2 < N_CHUNKS)
        def _fire_next():
            fire(g0 + 2, 0)
        consume(g0 + 1, 1)
        return 0

    lax.fori_loop(0, N_PAIRS, pair, 0)
    plsc.subcore_barrier()
    copy_out(s_out)

    fill(e_v2, 0, 0.0)
    zero_share(e_v2.at[0])
    fill(p_v2, 0, 1.0)
    plsc.subcore_barrier()

    def fire_cnt(g, b):
        base = base0 + g * CHUNK
        pltpu.async_copy(tgt_hbm.at[pl.ds(base, CHUNK)], idx_t2.at[b], sem)

    def consume_cnt(b):
        pltpu.make_async_copy(tgt_hbm.at[pl.ds(0, CHUNK)], idx_t2.at[b],
                              sem).wait()
        pltpu.sync_copy(p_v2.at[0], s_sh.at[idx_t2.at[b]], add=True)

    fire_cnt(0, 0)

    def pair_cnt(h, _):
        g0 = 2 * h
        fire_cnt(g0 + 1, 1)
        consume_cnt(0)

        @pl.when(g0 + 2 < N_CHUNKS)
        def _fire_next():
            fire_cnt(g0 + 2, 0)
        consume_cnt(1)
        return 0

    lax.fori_loop(0, N_PAIRS, pair_cnt, 0)
    plsc.subcore_barrier()
    copy_out(c_out)


def kernel(source_nodes, target_nodes, edge_features, node_features,
           timestamps, W1, b1, W2, b2):
    del timestamps
    w1a = W1[:D_FEAT]
    w1b = W1[D_FEAT:2 * D_FEAT]
    w1c = W1[2 * D_FEAT:]
    P, Q = _pq(node_features, w1a, w1b, b1)
    E = _e_proj(edge_features, w1c)
    s_partial, c_partial = _sc_aggregate(
        source_nodes.astype(jnp.int32), target_nodes.astype(jnp.int32),
        P, Q, E)
    return _final(s_partial, c_partial, W2, b2)


# speedup vs baseline: 4.5995x; 1.1083x over previous
import functools

import jax
import jax.numpy as jnp
from jax import lax
from jax.experimental import pallas as pl
from jax.experimental.pallas import tpu as pltpu
from jax.experimental.pallas import tpu_sc as plsc

N_NODES = 10000
N_EDGES = 320000
D_FEAT = 128
D_EDGE = 16
MSG_DIM = 128

NC = 2
NS = 16
LANES = 16

EDGES_PER_TILE = N_EDGES // (NC * NS)
CHUNK = 40
N_CHUNKS = EDGES_PER_TILE // CHUNK
NBUF = 3
ROWS_PER_TILE = 624
REM_ROWS = N_NODES - NS * ROWS_PER_TILE


def _pq_body(nf_ref, wa_ref, wb_ref, b1_ref, p_ref, q_ref):
    nf = nf_ref[...]
    p_ref[...] = jnp.dot(nf, wa_ref[...], preferred_element_type=jnp.float32)
    q_ref[...] = (
        jnp.dot(nf, wb_ref[...], preferred_element_type=jnp.float32)
        + b1_ref[...][None, :]
    )


def _pq(node_features, w1a, w1b, b1):
    blk = 2000
    grid = (N_NODES // blk,)
    return pl.pallas_call(
        _pq_body,
        grid=grid,
        in_specs=[
            pl.BlockSpec((blk, D_FEAT), lambda i: (i, 0)),
            pl.BlockSpec((D_FEAT, MSG_DIM), lambda i: (0, 0)),
            pl.BlockSpec((D_FEAT, MSG_DIM), lambda i: (0, 0)),
            pl.BlockSpec((MSG_DIM,), lambda i: (0,)),
        ],
        out_specs=[
            pl.BlockSpec((blk, MSG_DIM), lambda i: (i, 0)),
            pl.BlockSpec((blk, MSG_DIM), lambda i: (i, 0)),
        ],
        out_shape=[
            jax.ShapeDtypeStruct((N_NODES, MSG_DIM), jnp.float32),
            jax.ShapeDtypeStruct((N_NODES, MSG_DIM), jnp.float32),
        ],
    )(node_features, w1a, w1b, b1)


def _e_body(ef_ref, wc_ref, e_ref):
    e_ref[...] = jnp.dot(
        ef_ref[...], wc_ref[...], preferred_element_type=jnp.float32
    )


def _e_proj(edge_features, w1c):
    blk = 8000
    grid = (N_EDGES // blk,)
    return pl.pallas_call(
        _e_body,
        grid=grid,
        in_specs=[
            pl.BlockSpec((blk, D_EDGE), lambda i: (i, 0)),
            pl.BlockSpec((D_EDGE, MSG_DIM), lambda i: (0, 0)),
        ],
        out_specs=pl.BlockSpec((blk, MSG_DIM), lambda i: (i, 0)),
        out_shape=jax.ShapeDtypeStruct((N_EDGES, MSG_DIM), jnp.float32),
    )(edge_features, w1c)


def _final_body(s0_ref, s1_ref, c0_ref, c1_ref, w2_ref, b2_ref, out_ref):
    s = s0_ref[...] + s1_ref[...]
    c = c0_ref[:, 0] + c1_ref[:, 0]
    mean = s / jnp.maximum(c, 1.0)[:, None]
    out = jnp.dot(mean, w2_ref[...], preferred_element_type=jnp.float32)
    out_ref[...] = out + jnp.where(c > 0.0, 1.0, 0.0)[:, None] * b2_ref[...][None, :]


def _final(s_partial, c_partial, W2, b2):
    blk = 2000
    grid = (N_NODES // blk,)
    return pl.pallas_call(
        _final_body,
        grid=grid,
        in_specs=[
            pl.BlockSpec((blk, MSG_DIM), lambda i: (i, 0)),
            pl.BlockSpec((blk, MSG_DIM), lambda i: (i, 0)),
            pl.BlockSpec((blk, MSG_DIM), lambda i: (i, 0)),
            pl.BlockSpec((blk, MSG_DIM), lambda i: (i, 0)),
            pl.BlockSpec((MSG_DIM, MSG_DIM), lambda i: (0, 0)),
            pl.BlockSpec((MSG_DIM,), lambda i: (0,)),
        ],
        out_specs=pl.BlockSpec((blk, MSG_DIM), lambda i: (i, 0)),
        out_shape=jax.ShapeDtypeStruct((N_NODES, MSG_DIM), jnp.float32),
    )(
        s_partial[:N_NODES], s_partial[N_NODES:],
        c_partial[:N_NODES], c_partial[N_NODES:],
        W2, b2,
    )


@functools.partial(
    pl.kernel,
    out_type=[
        jax.ShapeDtypeStruct((NC * N_NODES, MSG_DIM), jnp.float32),
        jax.ShapeDtypeStruct((NC * N_NODES, MSG_DIM), jnp.float32),
    ],
    mesh=plsc.VectorSubcoreMesh(core_axis_name="c", subcore_axis_name="s"),
    scratch_types=[
        pltpu.VMEM((NBUF, CHUNK), jnp.int32),
        pltpu.VMEM((NBUF, CHUNK), jnp.int32),
        pltpu.VMEM((NBUF, CHUNK, MSG_DIM), jnp.float32),
        pltpu.VMEM((NBUF, CHUNK, MSG_DIM), jnp.float32),
        pltpu.VMEM((NBUF, CHUNK, MSG_DIM), jnp.float32),
        pltpu.VMEM_SHARED((N_NODES, MSG_DIM), jnp.float32),
        pltpu.SemaphoreType.DMA((2,)),
    ],
)
def _sc_aggregate(src_hbm, tgt_hbm, p_hbm, q_hbm, e_hbm, s_out, c_out,
                  idx_s2, idx_t2, p_v2, q_v2, e_v2, s_sh, sems):
    gsem = sems.at[0]
    ssem = sems.at[1]
    cid = lax.axis_index("c")
    sid = lax.axis_index("s")

    def fill(ref3, b, val):
        @plsc.parallel_loop(0, CHUNK, unroll=4)
        def body(r):
            for k in range(MSG_DIM // LANES):
                ref3[b, r, pl.ds(k * LANES, LANES)] = jnp.full(
                    (LANES,), val, jnp.float32)

    row0 = sid * ROWS_PER_TILE
    rem0 = NS * ROWS_PER_TILE

    def zero_share(zbuf):
        for j in range(15):
            pltpu.sync_copy(zbuf, s_sh.at[pl.ds(row0 + j * CHUNK, CHUNK)])
        pltpu.sync_copy(zbuf.at[pl.ds(0, 24)], s_sh.at[pl.ds(row0 + 600, 24)])

        @pl.when(sid == NS - 1)
        def _zero_rem():
            pltpu.sync_copy(zbuf.at[pl.ds(0, REM_ROWS)],
                            s_sh.at[pl.ds(rem0, REM_ROWS)])

    def copy_out(dst):
        out0 = cid * N_NODES + row0
        pltpu.sync_copy(s_sh.at[pl.ds(row0, ROWS_PER_TILE)],
                        dst.at[pl.ds(out0, ROWS_PER_TILE)])

        @pl.when(sid == NS - 1)
        def _copy_rem():
            orem = cid * N_NODES + rem0
            pltpu.sync_copy(s_sh.at[pl.ds(rem0, REM_ROWS)],
                            dst.at[pl.ds(orem, REM_ROWS)])

    base0 = (cid * NS + sid) * EDGES_PER_TILE

    fill(e_v2, 0, 0.0)
    zero_share(e_v2.at[0])
    plsc.subcore_barrier()

    def drain_scatter(b):
        pltpu.make_async_copy(p_hbm.at[pl.ds(0, CHUNK)], p_v2.at[b],
                              ssem).wait()

    def fire(g, b):
        base = base0 + g * CHUNK
        pltpu.sync_copy(src_hbm.at[pl.ds(base, CHUNK)], idx_s2.at[b])
        pltpu.sync_copy(tgt_hbm.at[pl.ds(base, CHUNK)], idx_t2.at[b])
        pltpu.async_copy(p_hbm.at[idx_s2.at[b]], p_v2.at[b], gsem)
        pltpu.async_copy(q_hbm.at[idx_t2.at[b]], q_v2.at[b], gsem)
        pltpu.async_copy(e_hbm.at[pl.ds(base, CHUNK)], e_v2.at[b], gsem)

    def drain(b):
        pltpu.make_async_copy(p_hbm.at[pl.ds(0, CHUNK)], p_v2.at[b], gsem).wait()
        pltpu.make_async_copy(q_hbm.at[pl.ds(0, CHUNK)], q_v2.at[b], gsem).wait()
        pltpu.make_async_copy(e_hbm.at[pl.ds(0, CHUNK)], e_v2.at[b], gsem).wait()

    def consume(g, b):
        drain(b)

        @plsc.parallel_loop(0, CHUNK, unroll=4)
        def row(r):
            for k in range(MSG_DIM // LANES):
                sl = pl.ds(k * LANES, LANES)
                p_v2[b, r, sl] = jnp.maximum(
                    p_v2[b, r, sl] + q_v2[b, r, sl] + e_v2[b, r, sl], 0.0)

        pltpu.async_copy(p_v2.at[b], s_sh.at[idx_t2.at[b]], ssem, add=True)

    fire(0, 0)
    fire(1, 1)

    def body(g, _):
        b = lax.rem(g, NBUF)
        nb = lax.rem(g + 2, NBUF)
        consume(g, b)

        @pl.when(g + 2 < N_CHUNKS)
        def _prep_next():
            @pl.when(g >= 1)
            def _drain_prev():
                drain_scatter(nb)
            fire(g + 2, nb)
        return 0

    lax.fori_loop(0, N_CHUNKS, body, 0)
    for _ in range(NBUF):
        drain_scatter(0)
    plsc.subcore_barrier()
    copy_out(s_out)

    fill(e_v2, 0, 0.0)
    zero_share(e_v2.at[0])
    fill(p_v2, 0, 1.0)
    plsc.subcore_barrier()

    def fire_cnt(g, b):
        base = base0 + g * CHUNK
        pltpu.async_copy(tgt_hbm.at[pl.ds(base, CHUNK)], idx_t2.at[b], gsem)

    def consume_cnt(b):
        pltpu.make_async_copy(tgt_hbm.at[pl.ds(0, CHUNK)], idx_t2.at[b],
                              gsem).wait()
        pltpu.async_copy(p_v2.at[0], s_sh.at[idx_t2.at[b]], ssem, add=True)

    fire_cnt(0, 0)
    fire_cnt(1, 1)

    def body_cnt(g, _):
        b = lax.rem(g, NBUF)
        nb = lax.rem(g + 2, NBUF)
        consume_cnt(b)

        @pl.when(g + 2 < N_CHUNKS)
        def _prep_next():
            @pl.when(g >= 1)
            def _drain_prev():
                drain_scatter(nb)
            fire_cnt(g + 2, nb)
        return 0

    lax.fori_loop(0, N_CHUNKS, body_cnt, 0)
    for _ in range(NBUF):
        drain_scatter(0)
    plsc.subcore_barrier()
    copy_out(c_out)


def kernel(source_nodes, target_nodes, edge_features, node_features,
           timestamps, W1, b1, W2, b2):
    del timestamps
    w1a = W1[:D_FEAT]
    w1b = W1[D_FEAT:2 * D_FEAT]
    w1c = W1[2 * D_FEAT:]
    P, Q = _pq(node_features, w1a, w1b, b1)
    E = _e_proj(edge_features, w1c)
    s_partial, c_partial = _sc_aggregate(
        source_nodes.astype(jnp.int32), target_nodes.astype(jnp.int32),
        P, Q, E)
    return _final(s_partial, c_partial, W2, b2)
